# baseline (speedup 1.0000x reference)
import jax
import jax.numpy as jnp
from jax import lax
from jax.experimental import pallas as pl
from jax.experimental.pallas import tpu as pltpu

_B, _R, _L = 4, 65536, 3
_LANES = 128
_ROWS = _B * _R // _LANES
_BLK = 512
_GRID = _ROWS // _BLK


def _loss_body(ro, rg, lo, lt, mg, mvo, out, acc):
    p = pl.program_id(0)
    i = pl.program_id(1)
    first = (p == 0) & (i == 0)

    @pl.when(first)
    def _():
        for q in range(4):
            acc[q] = 0.0

    mgf = mg[...].astype(jnp.float32)
    c = mvo[0].astype(jnp.int32)
    mvf = (c & 1).astype(jnp.float32)
    mof = ((c >> 1) & 1).astype(jnp.float32)

    x = 10.0 * (lt[0] - lo[0])
    bce = jnp.maximum(x, 0.0) - x * mgf + jnp.log1p(jnp.exp(-jnp.abs(x)))
    moo = mvf * (1.0 - mof * mgf)
    l1 = jnp.abs(ro[0] - rg[0])

    acc[0] += jnp.sum(l1 * mgf)
    acc[1] += jnp.sum(mgf)
    acc[2] += jnp.sum(bce * moo)
    acc[3] += jnp.sum(moo)

    last = (p == _L - 1) & (i == _GRID - 1)

    @pl.when(last)
    def _():
        loss_rgb = acc[0] / acc[1]
        loss_mask = (acc[2] / acc[3]) / 10.0
        out[...] = jnp.full((1, 1), loss_rgb + 100.0 * loss_mask,
                            dtype=jnp.float32)


@jax.jit
def _loss(ro, rg, lo, lt, mg, mvo):
    plane_spec = pl.BlockSpec((1, _BLK, _LANES), lambda p, i: (p, i, 0))
    mask_spec = pl.BlockSpec((_BLK, _LANES), lambda p, i: (i, 0))
    parts = pl.pallas_call(
        _loss_body,
        grid=(_L, _GRID),
        in_specs=[plane_spec] * 4 + [mask_spec, plane_spec],
        out_specs=pl.BlockSpec((1, 1), lambda p, i: (0, 0)),
        out_shape=jax.ShapeDtypeStruct((1, 1), jnp.float32),
        scratch_shapes=[pltpu.SMEM((4,), jnp.float32)],
        compiler_params=pltpu.CompilerParams(
            dimension_semantics=("arbitrary", "arbitrary")),
    )(ro, rg, lo, lt, mg, mvo)
    return parts[0, 0]


def _planar(x):
    return (x.reshape(_B, _R // _LANES, _LANES, _L)
            .transpose(3, 1, 0, 2)
            .reshape(_L, _ROWS, _LANES))


def _rows2d(m):
    return (m.reshape(_B, _R // _LANES, _LANES)
            .transpose(1, 0, 2)
            .reshape(_ROWS, _LANES))


def kernel(rgb_output, rgb_gt, level_output, level_target, mask_gt,
           mask_valid, mask_output, iteration):
    mvo = mask_valid.astype(jnp.int8) | (mask_output.astype(jnp.int8) << 1)
    return _loss(_planar(rgb_output), _planar(rgb_gt),
                 _planar(level_output), _planar(level_target),
                 _rows2d(mask_gt.astype(jnp.int8)), _planar(mvo))


# speedup vs baseline: 1.1546x; 1.1546x over previous
import jax
import jax.numpy as jnp
from jax import lax
from jax.experimental import pallas as pl
from jax.experimental.pallas import tpu as pltpu

_B, _R, _L = 4, 65536, 3
_LANES = 128
_ROWS = _B * _R // _LANES
_BLK = 1024
_GRID = _ROWS // _BLK


def _loss_body(ro, rg, lo, lt, mg, mvo, out, acc):
    p = pl.program_id(0)
    i = pl.program_id(1)
    first = (p == 0) & (i == 0)

    @pl.when(first)
    def _():
        for q in range(4):
            acc[q] = 0.0

    mgf = mg[...].astype(jnp.float32)
    c = mvo[0].astype(jnp.int32)
    mvf = (c & 1).astype(jnp.float32)
    mof = ((c >> 1) & 1).astype(jnp.float32)

    x = 10.0 * (lt[0] - lo[0])
    bce = jnp.maximum(x, 0.0) - x * mgf + jnp.log1p(jnp.exp(-jnp.abs(x)))
    moo = mvf * (1.0 - mof * mgf)
    l1 = jnp.abs(ro[0] - rg[0])

    acc[0] += jnp.sum(l1 * mgf)
    acc[1] += jnp.sum(mgf)
    acc[2] += jnp.sum(bce * moo)
    acc[3] += jnp.sum(moo)

    last = (p == _L - 1) & (i == _GRID - 1)

    @pl.when(last)
    def _():
        loss_rgb = acc[0] / acc[1]
        loss_mask = (acc[2] / acc[3]) / 10.0
        out[...] = jnp.full((1, 1), loss_rgb + 100.0 * loss_mask,
                            dtype=jnp.float32)


@jax.jit
def _loss(ro, rg, lo, lt, mg, mvo):
    plane_spec = pl.BlockSpec((1, _BLK, _LANES), lambda p, i: (p, i, 0))
    mask_spec = pl.BlockSpec((_BLK, _LANES), lambda p, i: (i, 0))
    parts = pl.pallas_call(
        _loss_body,
        grid=(_L, _GRID),
        in_specs=[plane_spec] * 4 + [mask_spec, plane_spec],
        out_specs=pl.BlockSpec((1, 1), lambda p, i: (0, 0)),
        out_shape=jax.ShapeDtypeStruct((1, 1), jnp.float32),
        scratch_shapes=[pltpu.SMEM((4,), jnp.float32)],
        compiler_params=pltpu.CompilerParams(
            dimension_semantics=("arbitrary", "arbitrary")),
    )(ro, rg, lo, lt, mg, mvo)
    return parts[0, 0]


def _planar(x):
    return (x.reshape(_B, _R // _LANES, _LANES, _L)
            .transpose(3, 1, 0, 2)
            .reshape(_L, _ROWS, _LANES))


def _rows2d(m):
    return (m.reshape(_B, _R // _LANES, _LANES)
            .transpose(1, 0, 2)
            .reshape(_ROWS, _LANES))


def kernel(rgb_output, rgb_gt, level_output, level_target, mask_gt,
           mask_valid, mask_output, iteration):
    mvo = mask_valid.astype(jnp.int8) | (mask_output.astype(jnp.int8) << 1)
    return _loss(_planar(rgb_output), _planar(rgb_gt),
                 _planar(level_output), _planar(level_target),
                 _rows2d(mask_gt.astype(jnp.int8)), _planar(mvo))


# speedup vs baseline: 1.1595x; 1.0042x over previous
import jax
import jax.numpy as jnp
from jax.experimental import pallas as pl
from jax.experimental.pallas import tpu as pltpu

_B, _R, _L = 4, 65536, 3
_LANES = 128
_ROWS = _B * _R // _LANES
_BLK = 1024
_GRID = _ROWS // _BLK


def _loss_body(ro, rg, lo, lt, mg, mvo, out, acc):
    p = pl.program_id(0)
    i = pl.program_id(1)
    first = (p == 0) & (i == 0)

    @pl.when(first)
    def _():
        for q in range(4):
            acc[q] = 0.0

    mgf = mg[...].astype(jnp.float32)
    c = mvo[0].astype(jnp.int32)
    mvf = (c & 1).astype(jnp.float32)
    mof = ((c >> 1) & 1).astype(jnp.float32)

    x = 10.0 * (lt[0] - lo[0])
    bce = jnp.maximum(x, 0.0) - x * mgf + jnp.log1p(jnp.exp(-jnp.abs(x)))
    moo = mvf * (1.0 - mof * mgf)
    l1 = jnp.abs(ro[0] - rg[0])

    acc[0] += jnp.sum(l1 * mgf)
    acc[1] += jnp.sum(mgf)
    acc[2] += jnp.sum(bce * moo)
    acc[3] += jnp.sum(moo)

    last = (p == _L - 1) & (i == _GRID - 1)

    @pl.when(last)
    def _():
        loss_rgb = acc[0] / acc[1]
        loss_mask = (acc[2] / acc[3]) / 10.0
        out[...] = jnp.full((1, 1), loss_rgb + 100.0 * loss_mask,
                            dtype=jnp.float32)


@jax.jit
def _loss(ro, rg, lo, lt, mg, mvo):
    plane_spec = pl.BlockSpec((1, _BLK, _LANES), lambda p, i: (p, i, 0))
    mask_spec = pl.BlockSpec((_BLK, _LANES), lambda p, i: (i, 0))
    parts = pl.pallas_call(
        _loss_body,
        grid=(_L, _GRID),
        in_specs=[plane_spec] * 4 + [mask_spec, plane_spec],
        out_specs=pl.BlockSpec((1, 1), lambda p, i: (0, 0)),
        out_shape=jax.ShapeDtypeStruct((1, 1), jnp.float32),
        scratch_shapes=[pltpu.SMEM((4,), jnp.float32)],
        compiler_params=pltpu.CompilerParams(
            dimension_semantics=("arbitrary", "arbitrary")),
    )(ro, rg, lo, lt, mg, mvo)
    return parts[0, 0]


def _planar(x):
    return (x.reshape(_B, _R // _LANES, _LANES, _L)
            .transpose(3, 1, 0, 2)
            .reshape(_L, _ROWS, _LANES))


def _rows2d(m):
    return (m.reshape(_B, _R // _LANES, _LANES)
            .transpose(1, 0, 2)
            .reshape(_ROWS, _LANES))


def kernel(rgb_output, rgb_gt, level_output, level_target, mask_gt,
           mask_valid, mask_output, iteration):
    mvo = mask_valid.astype(jnp.int8) | (mask_output.astype(jnp.int8) << 1)
    return _loss(_planar(rgb_output), _planar(rgb_gt),
                 _planar(level_output), _planar(level_target),
                 _rows2d(mask_gt.astype(jnp.int8)), _planar(mvo))
